# baseline (speedup 1.0000x reference)
import functools

import jax
import jax.numpy as jnp
from jax import lax
from jax.experimental import pallas as pl
from jax.experimental.pallas import tpu as pltpu
from jax.experimental.pallas import tpu_sc as plsc

NC = 2
NS = 16
NW = NC * NS
CH = 128


def _ln_relu_body(x_ref, g_ref, b_ref, o_ref):
    xb = x_ref[...]
    mean = jnp.mean(xb, axis=-1, keepdims=True)
    var = jnp.mean((xb - mean) ** 2, axis=-1, keepdims=True)
    h = (xb - mean) * lax.rsqrt(var + 1e-5) * g_ref[...] + b_ref[...]
    o_ref[...] = jnp.maximum(h, 0.0)


def _ln_relu(x, gamma, beta, blk):
    n, d = x.shape
    return pl.pallas_call(
        _ln_relu_body,
        grid=(n // blk,),
        in_specs=[
            pl.BlockSpec((blk, d), lambda i: (i, 0)),
            pl.BlockSpec((1, d), lambda i: (0, 0)),
            pl.BlockSpec((1, d), lambda i: (0, 0)),
        ],
        out_specs=pl.BlockSpec((blk, d), lambda i: (i, 0)),
        out_shape=jax.ShapeDtypeStruct((n, d), jnp.float32),
    )(x, gamma.reshape(1, d), beta.reshape(1, d))


def _make_sc_agg(n, d, ntab, n_chunks, per_w):
    mesh = plsc.VectorSubcoreMesh(
        core_axis_name="c", subcore_axis_name="s", num_cores=NC, num_subcores=NS
    )
    zstripe = ntab // NS

    @functools.partial(
        pl.kernel,
        out_type=(
            jax.ShapeDtypeStruct((NC, ntab, d), jnp.float32),
            jax.ShapeDtypeStruct((NC, ntab, d), jnp.float32),
        ),
        mesh=mesh,
        scratch_types=[
            pltpu.VMEM_SHARED((ntab, d), jnp.float32),
            pltpu.VMEM((CH,), jnp.int32),
            pltpu.VMEM((CH,), jnp.int32),
            pltpu.VMEM((CH,), jnp.int32),
            pltpu.VMEM((CH,), jnp.int32),
            pltpu.VMEM((CH, d), jnp.float32),
            pltpu.VMEM((CH, d), jnp.float32),
            pltpu.SemaphoreType.DMA,
            pltpu.SemaphoreType.DMA,
            pltpu.SemaphoreType.DMA,
            pltpu.SemaphoreType.DMA,
        ],
    )
    def sc_agg(h_hbm, src_hbm, dst_hbm,
               agg_out, deg_out, tab_sh, ib0, ib1, ib2, ib3,
               rows0, rows1, sm0, sm1, sm2, sm3):
        c = lax.axis_index("c")
        s = lax.axis_index("s")
        wid = c * NS + s
        z0 = s * zstripe
        g2 = n_chunks // 2
        g4 = n_chunks // 4

        lane = lax.iota(jnp.int32, 16)
        zv = jnp.where(lane < 0, jnp.float32(1.0), jnp.float32(0.0))
        ov = jnp.where(lane >= 0, jnp.float32(1.0), jnp.float32(0.0))

        def fill_rows(rref, val):
            def body(i, carry):
                for j in range(d // 16):
                    rref[i, pl.ds(j * 16, 16)] = val
                return carry
            lax.fori_loop(0, CH, body, 0)

        def zero_table():
            for t in range(zstripe // CH):
                pltpu.sync_copy(rows0, tab_sh.at[pl.ds(z0 + t * CH, CH)])

        def copy_table(out3):
            pltpu.sync_copy(tab_sh.at[pl.ds(z0, zstripe)],
                            out3.at[c, pl.ds(z0, zstripe)])

        def load_idx(iref, arr, k):
            base = pl.multiple_of(wid * per_w + k * CH, CH)
            pltpu.sync_copy(arr.at[pl.ds(base, CH)], iref)

        def wait_scatter(rref, iref, sem):
            pltpu.make_async_copy(rref, tab_sh.at[iref], sem).wait()

        def fill_sentinel(iref):
            sv = jnp.full((16,), n, jnp.int32)
            for j in range(CH // 16):
                iref[pl.ds(j * 16, 16)] = sv

        fill_rows(rows0, zv)
        zero_table()
        plsc.subcore_barrier()

        load_idx(ib0, src_hbm, 0)
        load_idx(ib2, dst_hbm, 0)
        pltpu.async_copy(h_hbm.at[ib0], rows0, sm0)
        fill_sentinel(ib3)
        pltpu.async_copy(rows1, tab_sh.at[ib3], sm3, add=True)

        def group_a(g, carry):
            k0 = 2 * g
            wait_scatter(rows1, ib3, sm3)
            load_idx(ib1, src_hbm, k0 + 1)
            load_idx(ib3, dst_hbm, k0 + 1)
            pltpu.async_copy(h_hbm.at[ib1], rows1, sm1)
            pltpu.make_async_copy(h_hbm.at[ib0], rows0, sm0).wait()
            pltpu.async_copy(rows0, tab_sh.at[ib2], sm2, add=True)

            wait_scatter(rows0, ib2, sm2)
            load_idx(ib0, src_hbm, k0 + 2)
            load_idx(ib2, dst_hbm, k0 + 2)
            pltpu.async_copy(h_hbm.at[ib0], rows0, sm0)
            pltpu.make_async_copy(h_hbm.at[ib1], rows1, sm1).wait()
            pltpu.async_copy(rows1, tab_sh.at[ib3], sm3, add=True)
            return carry

        lax.fori_loop(0, g2, group_a, 0)
        wait_scatter(rows1, ib3, sm3)
        pltpu.make_async_copy(h_hbm.at[ib0], rows0, sm0).wait()
        plsc.subcore_barrier()
        copy_table(agg_out)
        plsc.subcore_barrier()

        fill_rows(rows0, zv)
        zero_table()
        plsc.subcore_barrier()
        fill_rows(rows0, ov)

        ibs = (ib0, ib1, ib2, ib3)
        sms = (sm0, sm1, sm2, sm3)
        for b in range(4):
            fill_sentinel(ibs[b])
            pltpu.async_copy(rows0, tab_sh.at[ibs[b]], sms[b], add=True)

        def group_b(g, carry):
            for b in range(4):
                wait_scatter(rows0, ibs[b], sms[b])
                load_idx(ibs[b], dst_hbm, 4 * g + b)
                pltpu.async_copy(rows0, tab_sh.at[ibs[b]], sms[b], add=True)
            return carry

        lax.fori_loop(0, g4, group_b, 0)
        for b in range(4):
            wait_scatter(rows0, ibs[b], sms[b])
        plsc.subcore_barrier()
        copy_table(deg_out)

    return sc_agg


def _final_body(x_ref, h_ref, a0_ref, a1_ref, d0_ref, d1_ref,
                wl_ref, wr_ref, bl_ref, o_ref):
    agg = a0_ref[...] + a1_ref[...]
    deg = jnp.maximum(d0_ref[...] + d1_ref[...], 1.0)
    am = agg / deg
    acc = jnp.dot(am, wl_ref[...], preferred_element_type=jnp.float32)
    acc += jnp.dot(h_ref[...], wr_ref[...], preferred_element_type=jnp.float32)
    o_ref[...] = acc + bl_ref[...] + x_ref[...]


def _final(x, h, agg_parts, deg_parts, w_l_t, w_r_t, b_l, blk):
    n, d = x.shape
    row = lambda i: (i, 0)
    full = lambda i: (0, 0)
    return pl.pallas_call(
        _final_body,
        grid=(n // blk,),
        in_specs=[
            pl.BlockSpec((blk, d), row),
            pl.BlockSpec((blk, d), row),
            pl.BlockSpec((blk, d), row),
            pl.BlockSpec((blk, d), row),
            pl.BlockSpec((blk, d), row),
            pl.BlockSpec((blk, d), row),
            pl.BlockSpec((d, d), full),
            pl.BlockSpec((d, d), full),
            pl.BlockSpec((1, d), full),
        ],
        out_specs=pl.BlockSpec((blk, d), row),
        out_shape=jax.ShapeDtypeStruct((n, d), jnp.float32),
    )(x, h, agg_parts[0], agg_parts[1], deg_parts[0], deg_parts[1],
      w_l_t, w_r_t, b_l.reshape(1, d))


def kernel(x, edge_index, ln_gamma, ln_beta, W_l, b_l, W_r):
    n, d = x.shape
    e = edge_index.shape[1]

    n_chunks = -(-(-(-e // (NW * CH))) // 4) * 4
    e_pad = n_chunks * NW * CH
    per_w = n_chunks * CH
    src = edge_index[0].astype(jnp.int32)
    dst = edge_index[1].astype(jnp.int32)
    pad = e_pad - e + CH
    src = jnp.concatenate([src, jnp.zeros((pad,), jnp.int32)])
    dst = jnp.concatenate([dst, jnp.full((pad,), n, jnp.int32)])

    ntab = -(-(n + 1) // (NS * CH)) * NS * CH

    h = _ln_relu(x, ln_gamma, ln_beta, blk=1000)
    agg_parts, deg_parts = _make_sc_agg(n, d, ntab, n_chunks, per_w)(
        h, src, dst)
    return _final(x, h, agg_parts, deg_parts,
                  W_l.T, W_r.T, b_l, blk=1000)


# speedup vs baseline: 1.0618x; 1.0618x over previous
import functools

import jax
import jax.numpy as jnp
from jax import lax
from jax.experimental import pallas as pl
from jax.experimental.pallas import tpu as pltpu
from jax.experimental.pallas import tpu_sc as plsc

NC = 2
NS = 16
NW = NC * NS
CH = 128
BLK = 8


def _ln_relu_body(x_ref, g_ref, b_ref, o_ref):
    xb = x_ref[...]
    mean = jnp.mean(xb, axis=-1, keepdims=True)
    var = jnp.mean((xb - mean) ** 2, axis=-1, keepdims=True)
    h = (xb - mean) * lax.rsqrt(var + 1e-5) * g_ref[...] + b_ref[...]
    o_ref[...] = jnp.maximum(h, 0.0)


def _ln_relu(x, gamma, beta, blk):
    n, d = x.shape
    return pl.pallas_call(
        _ln_relu_body,
        grid=(n // blk,),
        in_specs=[
            pl.BlockSpec((blk, d), lambda i: (i, 0)),
            pl.BlockSpec((1, d), lambda i: (0, 0)),
            pl.BlockSpec((1, d), lambda i: (0, 0)),
        ],
        out_specs=pl.BlockSpec((blk, d), lambda i: (i, 0)),
        out_shape=jax.ShapeDtypeStruct((n, d), jnp.float32),
    )(x, gamma.reshape(1, d), beta.reshape(1, d))


def _make_sc_agg(n, d, ntab, n_chunks):
    mesh = plsc.VectorSubcoreMesh(
        core_axis_name="c", subcore_axis_name="s", num_cores=NC, num_subcores=NS
    )
    zstripe = ntab // NS

    @functools.partial(
        pl.kernel,
        out_type=(
            jax.ShapeDtypeStruct((NC, ntab, d), jnp.float32),
            jax.ShapeDtypeStruct((NC, ntab, d), jnp.float32),
        ),
        mesh=mesh,
        scratch_types=[
            pltpu.VMEM_SHARED((ntab, d), jnp.float32),
            pltpu.VMEM((BLK, CH), jnp.int32),
            pltpu.VMEM((BLK, CH), jnp.int32),
            pltpu.VMEM((CH, d), jnp.float32),
            pltpu.VMEM((CH, d), jnp.float32),
            pltpu.SemaphoreType.DMA,
            pltpu.SemaphoreType.DMA,
        ],
    )
    def sc_agg(h_hbm, src_hbm, dst_hbm,
               agg_out, deg_out, tab_sh, sib, dib,
               rows0, rows1, sm0, sm1):
        c = lax.axis_index("c")
        s = lax.axis_index("s")
        wid = c * NS + s
        z0 = s * zstripe
        ngrp = n_chunks // BLK

        lane = lax.iota(jnp.int32, 16)
        zv = jnp.where(lane < 0, jnp.float32(1.0), jnp.float32(0.0))
        ov = jnp.where(lane >= 0, jnp.float32(1.0), jnp.float32(0.0))

        def fill_rows(rref, val):
            def body(i, carry):
                for j in range(d // 16):
                    rref[i, pl.ds(j * 16, 16)] = val
                return carry
            lax.fori_loop(0, CH, body, 0)

        def zero_table():
            for t in range(zstripe // CH):
                pltpu.sync_copy(rows0, tab_sh.at[pl.ds(z0 + t * CH, CH)])

        def copy_table(out3):
            pltpu.sync_copy(tab_sh.at[pl.ds(z0, zstripe)],
                            out3.at[c, pl.ds(z0, zstripe)])

        def load_blk(iref, arr2, g):
            r0 = pl.multiple_of((wid * n_chunks + g * BLK), BLK)
            pltpu.sync_copy(arr2.at[pl.ds(r0, BLK)], iref)

        fill_rows(rows0, zv)
        fill_rows(rows1, zv)
        zero_table()
        plsc.subcore_barrier()

        rbufs = (rows0, rows1)
        sms = (sm0, sm1)

        def group_a(g, carry):
            load_blk(sib, src_hbm, g)
            load_blk(dib, dst_hbm, g)
            pltpu.async_copy(h_hbm.at[sib.at[0]], rows0, sm0)
            for t in range(BLK):
                b = t % 2
                if t + 1 < BLK:
                    pltpu.async_copy(h_hbm.at[sib.at[t + 1]],
                                     rbufs[1 - b], sms[1 - b])
                pltpu.make_async_copy(h_hbm.at[sib.at[t]],
                                      rbufs[b], sms[b]).wait()
                pltpu.sync_copy(rbufs[b], tab_sh.at[dib.at[t]], add=True)
            return carry

        lax.fori_loop(0, ngrp, group_a, 0)
        plsc.subcore_barrier()
        copy_table(agg_out)
        plsc.subcore_barrier()

        fill_rows(rows0, zv)
        zero_table()
        plsc.subcore_barrier()
        fill_rows(rows0, ov)

        def group_b(g, carry):
            load_blk(dib, dst_hbm, g)
            for t in range(BLK):
                pltpu.sync_copy(rows0, tab_sh.at[dib.at[t]], add=True)
            return carry

        lax.fori_loop(0, ngrp, group_b, 0)
        plsc.subcore_barrier()
        copy_table(deg_out)

    return sc_agg


def _final_body(x_ref, h_ref, a0_ref, a1_ref, d0_ref, d1_ref,
                wl_ref, wr_ref, bl_ref, o_ref):
    agg = a0_ref[...] + a1_ref[...]
    deg = jnp.maximum(d0_ref[...] + d1_ref[...], 1.0)
    am = agg / deg
    acc = jnp.dot(am, wl_ref[...], preferred_element_type=jnp.float32)
    acc += jnp.dot(h_ref[...], wr_ref[...], preferred_element_type=jnp.float32)
    o_ref[...] = acc + bl_ref[...] + x_ref[...]


def _final(x, h, agg_parts, deg_parts, w_l_t, w_r_t, b_l, blk):
    n, d = x.shape
    row = lambda i: (i, 0)
    full = lambda i: (0, 0)
    return pl.pallas_call(
        _final_body,
        grid=(n // blk,),
        in_specs=[
            pl.BlockSpec((blk, d), row),
            pl.BlockSpec((blk, d), row),
            pl.BlockSpec((blk, d), row),
            pl.BlockSpec((blk, d), row),
            pl.BlockSpec((blk, d), row),
            pl.BlockSpec((blk, d), row),
            pl.BlockSpec((d, d), full),
            pl.BlockSpec((d, d), full),
            pl.BlockSpec((1, d), full),
        ],
        out_specs=pl.BlockSpec((blk, d), row),
        out_shape=jax.ShapeDtypeStruct((n, d), jnp.float32),
    )(x, h, agg_parts[0], agg_parts[1], deg_parts[0], deg_parts[1],
      w_l_t, w_r_t, b_l.reshape(1, d))


def kernel(x, edge_index, ln_gamma, ln_beta, W_l, b_l, W_r):
    n, d = x.shape
    e = edge_index.shape[1]

    n_chunks = -(-(-(-e // (NW * CH))) // BLK) * BLK
    e_pad = n_chunks * NW * CH
    src = edge_index[0].astype(jnp.int32)
    dst = edge_index[1].astype(jnp.int32)
    pad = e_pad - e
    if pad:
        src = jnp.concatenate([src, jnp.zeros((pad,), jnp.int32)])
        dst = jnp.concatenate([dst, jnp.full((pad,), n, jnp.int32)])
    src = src.reshape(e_pad // CH, CH)
    dst = dst.reshape(e_pad // CH, CH)

    ntab = -(-(n + 1) // (NS * CH)) * NS * CH

    h = _ln_relu(x, ln_gamma, ln_beta, blk=1000)
    agg_parts, deg_parts = _make_sc_agg(n, d, ntab, n_chunks)(
        h, src, dst)
    return _final(x, h, agg_parts, deg_parts,
                  W_l.T, W_r.T, b_l, blk=1000)
